# baseline (speedup 1.0000x reference)
import math

import jax
import jax.numpy as jnp
from jax import lax
from jax.experimental import pallas as pl
from jax.experimental.pallas import tpu as pltpu
from jax.experimental.pallas import tpu_sc as plsc

D_MODEL = 64
SCALE = math.exp(64)

_INFO = plsc.get_sparse_core_info()
NC = _INFO.num_cores
NS = _INFO.num_subcores
NW = NC * NS
SUB = 128
NSUB = 4
CHUNK = SUB * NSUB


def _make_lookup(n_idx_rows: int):
    rows_per_w = n_idx_rows // NW
    chunks = rows_per_w // NSUB
    b = n_idx_rows * SUB

    mesh = plsc.VectorSubcoreMesh(core_axis_name="c", subcore_axis_name="s")

    @pl.kernel(
        out_type=jax.ShapeDtypeStruct((b, D_MODEL), jnp.float32),
        mesh=mesh,
        scratch_types=[
            pltpu.VMEM((NSUB, SUB), jnp.int32),
            pltpu.VMEM((CHUNK, D_MODEL), jnp.float32),
            pltpu.SemaphoreType.DMA,
        ],
        compiler_params=pltpu.CompilerParams(use_tc_tiling_on_sc=False),
    )
    def lookup(idx_hbm, table_hbm, out_hbm, idx_v, rows_v, sem):
        wid = lax.axis_index("s") * NC + lax.axis_index("c")
        row0 = wid * rows_per_w

        def chunk_body(ci, _):
            irow = row0 + ci * NSUB
            pltpu.sync_copy(idx_hbm.at[pl.ds(irow, NSUB)], idx_v)
            descs = [
                pltpu.async_copy(
                    table_hbm.at[idx_v.at[j]],
                    rows_v.at[pl.ds(j * SUB, SUB)],
                    sem,
                )
                for j in range(NSUB)
            ]
            for d in descs:
                d.wait()

            def scale_body(r, _):
                for c in range(D_MODEL // 16):
                    v = rows_v[r, pl.ds(c * 16, 16)]
                    rows_v[r, pl.ds(c * 16, 16)] = v * SCALE
                return 0

            lax.fori_loop(0, CHUNK, scale_body, 0, unroll=2)
            pltpu.sync_copy(rows_v, out_hbm.at[pl.ds(irow * SUB, CHUNK)])
            return 0

        lax.fori_loop(0, chunks, chunk_body, 0)

    return lookup


def kernel(x, table):
    orig_shape = x.shape
    idx = x.reshape(-1).astype(jnp.int32)
    n = idx.shape[0]
    assert n % (SUB * NSUB * NW) == 0
    idx2d = idx.reshape(n // SUB, SUB)
    out = _make_lookup(n // SUB)(idx2d, table)
    return out.reshape(*orig_shape, D_MODEL)


# speedup vs baseline: 1.0923x; 1.0923x over previous
import math

import jax
import jax.numpy as jnp
from jax import lax
from jax.experimental import pallas as pl
from jax.experimental.pallas import tpu as pltpu
from jax.experimental.pallas import tpu_sc as plsc

D_MODEL = 64
SCALE = math.exp(64)
LANES = 16

_INFO = plsc.get_sparse_core_info()
NC = _INFO.num_cores
NS = _INFO.num_subcores
NW = NC * NS
SUB = 128
NSUB = 2
CHUNK = SUB * NSUB


def _make_lookup(n_idx_rows: int):
    rows_per_w = n_idx_rows // NW
    chunks = rows_per_w // NSUB
    b = n_idx_rows * SUB
    assert chunks >= 4 and chunks % 2 == 0

    mesh = plsc.VectorSubcoreMesh(core_axis_name="c", subcore_axis_name="s")

    @pl.kernel(
        out_type=jax.ShapeDtypeStruct((b, D_MODEL), jnp.float32),
        mesh=mesh,
        scratch_types=[
            pltpu.VMEM((rows_per_w, SUB), jnp.int32),
            pltpu.VMEM((CHUNK, D_MODEL), jnp.float32),
            pltpu.VMEM((CHUNK, D_MODEL), jnp.float32),
            pltpu.VMEM((CHUNK, D_MODEL), jnp.float32),
            pltpu.VMEM((CHUNK, D_MODEL), jnp.float32),
            pltpu.SemaphoreType.DMA,
            pltpu.SemaphoreType.DMA,
            pltpu.SemaphoreType.DMA,
            pltpu.SemaphoreType.DMA,
        ],
        compiler_params=pltpu.CompilerParams(use_tc_tiling_on_sc=False),
    )
    def lookup(idx_hbm, table_hbm, out_hbm, idx_v, in0, in1, out0, out1,
               si0, si1, so0, so1):
        wid = lax.axis_index("s") * NC + lax.axis_index("c")
        irow0 = wid * rows_per_w
        orow0 = irow0 * SUB
        bufs = ((in0, out0, si0, so0), (in1, out1, si1, so1))

        def fire_gather(g, in_b, sem):
            for j in range(NSUB):
                pltpu.async_copy(
                    table_hbm.at[idx_v.at[g * NSUB + j]],
                    in_b.at[pl.ds(j * SUB, SUB)],
                    sem,
                )

        def wait_gather(in_b, sem):
            for j in range(NSUB):
                pltpu.make_async_copy(
                    table_hbm.at[idx_v.at[j]],
                    in_b.at[pl.ds(j * SUB, SUB)],
                    sem,
                ).wait()

        def fire_out(g, out_b, sem):
            pltpu.async_copy(
                out_b, out_hbm.at[pl.ds(orow0 + g * CHUNK, CHUNK)], sem)

        def wait_out(out_b, sem):
            pltpu.make_async_copy(
                out_b, out_hbm.at[pl.ds(orow0, CHUNK)], sem).wait()

        def scale(in_b, out_b):
            @plsc.parallel_loop(0, CHUNK, 1, unroll=4)
            def _(r):
                for c in range(D_MODEL // LANES):
                    sl = pl.ds(c * LANES, LANES)
                    out_b[r, sl] = in_b[r, sl] * SCALE

        pltpu.sync_copy(idx_hbm.at[pl.ds(irow0, rows_per_w)], idx_v)

        for bi in range(2):
            fire_gather(bi, bufs[bi][0], bufs[bi][2])

        for bi in range(2):
            in_b, out_b, si, so = bufs[bi]
            wait_gather(in_b, si)
            scale(in_b, out_b)
            fire_out(bi, out_b, so)
            fire_gather(bi + 2, in_b, si)

        def body(i, _):
            for bi in range(2):
                g = 2 + 2 * i + bi
                in_b, out_b, si, so = bufs[bi]
                wait_gather(in_b, si)
                wait_out(out_b, so)
                scale(in_b, out_b)
                fire_out(g, out_b, so)
                fire_gather(g + 2, in_b, si)
            return 0

        lax.fori_loop(0, (chunks - 4) // 2, body, 0)

        for bi in range(2):
            g = chunks - 2 + bi
            in_b, out_b, si, so = bufs[bi]
            wait_gather(in_b, si)
            wait_out(out_b, so)
            scale(in_b, out_b)
            fire_out(g, out_b, so)
        for bi in range(2):
            wait_out(bufs[bi][1], bufs[bi][3])

    return lookup


def kernel(x, table):
    orig_shape = x.shape
    idx = x.reshape(-1).astype(jnp.int32)
    n = idx.shape[0]
    assert n % (SUB * NSUB * NW) == 0
    idx2d = idx.reshape(n // SUB, SUB)
    out = _make_lookup(n // SUB)(idx2d, table)
    return out.reshape(*orig_shape, D_MODEL)
